# Initial kernel scaffold; baseline (speedup 1.0000x reference)
import functools

import jax
import jax.numpy as jnp
from jax import lax
from jax.experimental import pallas as pl
from jax.experimental.pallas import tpu as pltpu
from jax.experimental.pallas import tpu_sc as plsc

D = 32
ROW = 128
N_IDX_ROWS = 6400
NC = 2
NS = 16
NW = NC * NS
ROWS_PER_W = N_IDX_ROWS // NW
G = 10
STEPS = ROWS_PER_W // G


def _build():
    mesh = plsc.VectorSubcoreMesh(core_axis_name="c", subcore_axis_name="s")

    @functools.partial(
        pl.kernel,
        mesh=mesh,
        out_type=jax.ShapeDtypeStruct((N_IDX_ROWS, ROW, D), jnp.float32),
        scratch_types=[
            pltpu.VMEM((ROWS_PER_W, ROW), jnp.int32),
            pltpu.VMEM((2, G, ROW, D), jnp.float32),
            pltpu.SemaphoreType.DMA,
            pltpu.SemaphoreType.DMA,
        ],
    )
    def gather_kernel(idx_hbm, table_hbm, out_hbm, idx_v, rows_v, sem0, sem1):
        wid = lax.axis_index("s") * NC + lax.axis_index("c")
        base = wid * ROWS_PER_W
        pltpu.sync_copy(idx_hbm.at[pl.ds(base, ROWS_PER_W)], idx_v)

        def fire(c, buf, sem):
            for j in range(G):
                pltpu.async_copy(
                    table_hbm.at[idx_v.at[c * G + j]], rows_v.at[buf, j], sem)

        def drain_store(c, buf, sem):
            for j in range(G):
                pltpu.make_async_copy(
                    table_hbm.at[idx_v.at[c * G + j]], rows_v.at[buf, j],
                    sem).wait()
            pltpu.sync_copy(rows_v.at[buf],
                            out_hbm.at[pl.ds(base + c * G, G)])

        fire(0, 0, sem0)
        fire(1, 1, sem1)

        def body(p, carry):
            drain_store(2 * p, 0, sem0)
            fire(2 * p + 2, 0, sem0)
            drain_store(2 * p + 1, 1, sem1)
            fire(2 * p + 3, 1, sem1)
            return carry

        lax.fori_loop(0, STEPS // 2 - 1, body, 0)
        drain_store(STEPS - 2, 0, sem0)
        drain_store(STEPS - 1, 1, sem1)

    return gather_kernel


_GATHER = _build()


def kernel(item_id, table):
    b, h = item_id.shape
    idx = item_id.reshape(N_IDX_ROWS, ROW).astype(jnp.int32)
    out = _GATHER(idx, table)
    return out.reshape(b, h, D)


# speedup vs baseline: 1.3120x; 1.3120x over previous
import functools

import jax
import jax.numpy as jnp
from jax import lax
from jax.experimental import pallas as pl
from jax.experimental.pallas import tpu as pltpu
from jax.experimental.pallas import tpu_sc as plsc

D = 32
ROW = 128
N_IDX_ROWS = 6400
NC = 2
NS = 16
NW = NC * NS
ROWS_PER_W = N_IDX_ROWS // NW
G = 10
STEPS = ROWS_PER_W // G


def _build():
    mesh = plsc.VectorSubcoreMesh(core_axis_name="c", subcore_axis_name="s")

    @functools.partial(
        pl.kernel,
        mesh=mesh,
        compiler_params=pltpu.CompilerParams(use_tc_tiling_on_sc=False),
        out_type=jax.ShapeDtypeStruct((N_IDX_ROWS, ROW, D), jnp.float32),
        scratch_types=[
            pltpu.VMEM((ROWS_PER_W, ROW), jnp.int32),
            pltpu.VMEM((2, G, ROW, D), jnp.float32),
            pltpu.SemaphoreType.DMA,
            pltpu.SemaphoreType.DMA,
        ],
    )
    def gather_kernel(idx_hbm, table_hbm, out_hbm, idx_v, rows_v, sem0, sem1):
        wid = lax.axis_index("s") * NC + lax.axis_index("c")
        base = wid * ROWS_PER_W
        pltpu.sync_copy(idx_hbm.at[pl.ds(base, ROWS_PER_W)], idx_v)

        def fire(c, buf, sem):
            for j in range(G):
                pltpu.async_copy(
                    table_hbm.at[idx_v.at[c * G + j]], rows_v.at[buf, j], sem)

        def drain_store(c, buf, sem):
            for j in range(G):
                pltpu.make_async_copy(
                    table_hbm.at[idx_v.at[c * G + j]], rows_v.at[buf, j],
                    sem).wait()
            pltpu.sync_copy(rows_v.at[buf],
                            out_hbm.at[pl.ds(base + c * G, G)])

        fire(0, 0, sem0)
        fire(1, 1, sem1)

        def body(p, carry):
            drain_store(2 * p, 0, sem0)
            fire(2 * p + 2, 0, sem0)
            drain_store(2 * p + 1, 1, sem1)
            fire(2 * p + 3, 1, sem1)
            return carry

        lax.fori_loop(0, STEPS // 2 - 1, body, 0)
        drain_store(STEPS - 2, 0, sem0)
        drain_store(STEPS - 1, 1, sem1)

    return gather_kernel


_GATHER = _build()


def kernel(item_id, table):
    b, h = item_id.shape
    idx = item_id.reshape(N_IDX_ROWS, ROW).astype(jnp.int32)
    out = _GATHER(idx, table)
    return out.reshape(b, h, D)


# speedup vs baseline: 1.8605x; 1.4181x over previous
import functools

import jax
import jax.numpy as jnp
from jax import lax
from jax.experimental import pallas as pl
from jax.experimental.pallas import tpu as pltpu
from jax.experimental.pallas import tpu_sc as plsc

D = 32
B = 16384
H = 50
N = B * H
NC = 2
NS = 16
NW = NC * NS
TBW = B // (128 * NW)
PANELS = H * TBW
BANK = 8
GROUPS = PANELS // BANK


def _transpose_panel(rows, out_v, i0, i1):
    def cbody(c, carry):
        v0 = rows[c, pl.ds(0, 16)]
        v1 = rows[c, pl.ds(16, 16)]
        plsc.store_scatter(out_v, [i0 + c], v0)
        plsc.store_scatter(out_v, [i1 + c], v1)
        return carry

    lax.fori_loop(0, 128, cbody, 0)


def _build():
    mesh = plsc.VectorSubcoreMesh(core_axis_name="c", subcore_axis_name="s")

    @functools.partial(
        pl.kernel,
        mesh=mesh,
        compiler_params=pltpu.CompilerParams(use_tc_tiling_on_sc=False,
                                             needs_layout_passes=False),
        out_type=jax.ShapeDtypeStruct((N * D,), jnp.float32),
        scratch_types=[
            pltpu.VMEM((H, 128 * TBW), jnp.int32),
            pltpu.VMEM((2 * BANK, 128, D), jnp.float32),
            pltpu.VMEM((128 * D,), jnp.float32),
            pltpu.VMEM((128 * D,), jnp.float32),
            pltpu.SemaphoreType.DMA,
            pltpu.SemaphoreType.DMA,
            pltpu.SemaphoreType.DMA,
            pltpu.SemaphoreType.DMA,
            pltpu.SemaphoreType.DMA,
        ],
    )
    def gather_kernel(idx_hbm, table_hbm, out_hbm, idx_v, rows_v, out_va,
                      out_vb, sga, sgb, sst0, sst1, sidx):
        wid = lax.axis_index("s") * NC + lax.axis_index("c")
        nbase = 128 * TBW * wid

        for h in range(H):
            pltpu.async_copy(
                idx_hbm.at[pl.ds(h * B + nbase, 128 * TBW)], idx_v.at[h],
                sidx)
        for h in range(H):
            pltpu.make_async_copy(
                idx_hbm.at[pl.ds(h * B + nbase, 128 * TBW)], idx_v.at[h],
                sidx).wait()

        ib = lax.iota(jnp.int32, 16)
        i0 = ib * 128
        i1 = i0 + (16 * 128)
        ssts = (sst0, sst1)
        obufs = (out_va, out_vb)

        def fire(p, buf, sem):
            h = p // TBW
            t = p % TBW
            pltpu.async_copy(
                table_hbm.at[idx_v.at[h, pl.ds(t * 128, 128)]],
                rows_v.at[buf], sem)

        def wait_gather(p, buf, sem):
            h = p // TBW
            t = p % TBW
            pltpu.make_async_copy(
                table_hbm.at[idx_v.at[h, pl.ds(t * 128, 128)]],
                rows_v.at[buf], sem).wait()

        def store(p, ob):
            h = p // TBW
            tb = TBW * wid + (p % TBW)
            for ta in range(4):
                ofs = ((h * 4 + ta) * (B // 128) + tb) * 1024
                pltpu.async_copy(obufs[ob].at[pl.ds(ta * 1024, 1024)],
                                 out_hbm.at[pl.ds(ofs, 1024)], ssts[ob])

        def wait_store(p, ob):
            h = p // TBW
            tb = TBW * wid + (p % TBW)
            for ta in range(4):
                ofs = ((h * 4 + ta) * (B // 128) + tb) * 1024
                pltpu.make_async_copy(
                    obufs[ob].at[pl.ds(ta * 1024, 1024)],
                    out_hbm.at[pl.ds(ofs, 1024)], ssts[ob]).wait()

        def process(p, buf, pipe):
            ob = buf % 2
            wait_gather(p, buf, sga if buf < BANK else sgb)

            if isinstance(pipe, int):
                if pipe >= 2:
                    wait_store(p - 2, ob)
            else:
                @pl.when(pipe >= 2)
                def _():
                    wait_store(p - 2, ob)

            _transpose_panel(rows_v.at[buf], obufs[ob], i0, i1)
            store(p, ob)

        for b in range(BANK):
            fire(b, b, sga)
        for b in range(BANK):
            fire(BANK + b, BANK + b, sgb)

        def body(gg, carry):
            ga = 2 * gg
            for b in range(BANK):
                process(ga * BANK + b, b, ga * BANK + b)

            @pl.when(ga + 2 <= GROUPS - 1)
            def _():
                for b in range(BANK):
                    fire((ga + 2) * BANK + b, b, sga)

            for b in range(BANK):
                process((ga + 1) * BANK + b, BANK + b,
                        (ga + 1) * BANK + b)

            @pl.when(ga + 3 <= GROUPS - 1)
            def _():
                for b in range(BANK):
                    fire((ga + 3) * BANK + b, BANK + b, sgb)

            return carry

        lax.fori_loop(0, (GROUPS - 1) // 2, body, 0)
        for b in range(BANK):
            process((GROUPS - 1) * BANK + b, b, (GROUPS - 1) * BANK + b)
        wait_store(PANELS - 2, 0)
        wait_store(PANELS - 1, 1)

    return gather_kernel


_GATHER = _build()


def kernel(item_id, table):
    idx_t = jnp.transpose(item_id).reshape(-1).astype(jnp.int32)
    out1d = _GATHER(idx_t, table)
    out5 = out1d.reshape(H, 4, B // 128, 8, 128)
    return out5.transpose(2, 4, 0, 1, 3).reshape(B, H, D)
